# Initial kernel scaffold; baseline (speedup 1.0000x reference)
import jax
import jax.numpy as jnp
from jax.experimental import pallas as pl


def kernel(x, table):
    raise NotImplementedError("write your pallas kernel here")


# speedup vs baseline: 1.0097x; 1.0097x over previous
import functools
import math

import numpy as np
import jax
import jax.numpy as jnp
from jax import lax
from jax.experimental import pallas as pl
from jax.experimental.pallas import tpu as pltpu
from jax.experimental.pallas import tpu_sc as plsc

D_MODEL = 128
MAX_SEQ = 8192
NC, NS = 2, 16
NW = NC * NS
LANES = 16
R = 128
SCALE = math.sqrt(float(D_MODEL))


def _make_pe_np(max_seq, d_model):
    position = np.arange(max_seq, dtype=np.float32)[:, None]
    div_term = np.exp(
        np.arange(0, d_model, 2, dtype=np.float32) * (-math.log(10000.0) / d_model))
    pe = np.zeros((max_seq, d_model), dtype=np.float32)
    pe[:, 0::2] = np.sin(position * div_term)
    pe[:, 1::2] = np.cos(position * div_term)
    return pe


@functools.cache
def _build(ntot, seq_len, d):
    assert ntot % NW == 0
    bpw = ntot // NW
    assert bpw % R == 0
    nch = bpw // R
    assert seq_len % bpw == 0 or bpw % seq_len == 0
    mesh = plsc.VectorSubcoreMesh(core_axis_name="c", subcore_axis_name="s")

    @functools.partial(
        pl.kernel,
        mesh=mesh,
        out_type=jax.ShapeDtypeStruct((ntot, d), jnp.float32),
        scratch_types=[
            pltpu.VMEM((nch, R), jnp.int32),
            pltpu.VMEM((2, R, d), jnp.float32),
            pltpu.VMEM((2, R, d), jnp.float32),
            pltpu.SemaphoreType.DMA,
            pltpu.SemaphoreType.DMA,
            pltpu.SemaphoreType.DMA,
            pltpu.SemaphoreType.DMA,
            pltpu.SemaphoreType.DMA,
            pltpu.SemaphoreType.DMA,
        ],
    )
    def emb_kernel(table_hbm, x_hbm, pe_hbm, out_hbm,
                   idx_v, rows_v, pe_v, g0, g1, p0, p1, o0, o1):
        gsem = (g0, g1)
        psem = (p0, p1)
        osem = (o0, o1)
        wid = lax.axis_index("s") * NC + lax.axis_index("c")
        base = wid * bpw
        s_base = lax.rem(base, seq_len)

        pltpu.sync_copy(x_hbm.at[wid], idx_v)

        gd = [None] * nch
        pd = [None] * nch
        od = [None] * nch
        gd[0] = pltpu.async_copy(table_hbm.at[idx_v.at[0]], rows_v.at[0], gsem[0])
        pd[0] = pltpu.async_copy(pe_hbm.at[pl.ds(s_base, R)], pe_v.at[0], psem[0])
        for c in range(nch):
            b = c & 1
            nb = b ^ 1
            if c + 1 < nch:
                if c >= 1:
                    od[c - 1].wait()
                gd[c + 1] = pltpu.async_copy(
                    table_hbm.at[idx_v.at[c + 1]], rows_v.at[nb], gsem[nb])
                pd[c + 1] = pltpu.async_copy(
                    pe_hbm.at[pl.ds(s_base + (c + 1) * R, R)], pe_v.at[nb], psem[nb])
            gd[c].wait()
            pd[c].wait()

            def comp(i, carry, _b=b):
                for j in range(d // LANES):
                    sl = pl.ds(j * LANES, LANES)
                    rows_v[_b, i, sl] = rows_v[_b, i, sl] * SCALE + pe_v[_b, i, sl]
                return carry

            lax.fori_loop(0, R, comp, 0)
            od[c] = pltpu.async_copy(
                rows_v.at[b], out_hbm.at[pl.ds(base + c * R, R)], osem[b])
        if nch >= 2:
            od[nch - 2].wait()
        od[nch - 1].wait()

    return emb_kernel


def kernel(x, table):
    batch, seq_len = x.shape
    d = table.shape[1]
    ntot = batch * seq_len
    pe = jnp.asarray(_make_pe_np(MAX_SEQ, d)[:seq_len])
    xr = x.reshape(NW, ntot // (NW * R), R).astype(jnp.int32)
    out = _build(ntot, seq_len, d)(table, xr, pe)
    return out.reshape(batch, seq_len, d)


# speedup vs baseline: 1.1281x; 1.1172x over previous
import functools
import math

import numpy as np
import jax
import jax.numpy as jnp
from jax import lax
from jax.experimental import pallas as pl
from jax.experimental.pallas import tpu as pltpu
from jax.experimental.pallas import tpu_sc as plsc

D_MODEL = 128
MAX_SEQ = 8192
NC, NS = 2, 16
NW = NC * NS
LANES = 16
R = 128
SCALE = math.sqrt(float(D_MODEL))


def _make_pe_np(max_seq, d_model):
    position = np.arange(max_seq, dtype=np.float32)[:, None]
    div_term = np.exp(
        np.arange(0, d_model, 2, dtype=np.float32) * (-math.log(10000.0) / d_model))
    pe = np.zeros((max_seq, d_model), dtype=np.float32)
    pe[:, 0::2] = np.sin(position * div_term)
    pe[:, 1::2] = np.cos(position * div_term)
    return pe


@functools.cache
def _build(batch, seq_len, d):
    assert seq_len % NW == 0
    spw = seq_len // NW
    assert spw % R == 0
    hpw = spw // R
    nch = batch * hpw
    mesh = plsc.VectorSubcoreMesh(core_axis_name="c", subcore_axis_name="s")

    @functools.partial(
        pl.kernel,
        mesh=mesh,
        out_type=jax.ShapeDtypeStruct((batch, seq_len, d), jnp.float32),
        scratch_types=[
            pltpu.VMEM((batch, spw), jnp.int32),
            pltpu.VMEM((2, R, d), jnp.float32),
            pltpu.VMEM((spw, d), jnp.float32),
            pltpu.SemaphoreType.DMA,
            pltpu.SemaphoreType.DMA,
            pltpu.SemaphoreType.DMA,
            pltpu.SemaphoreType.DMA,
            pltpu.SemaphoreType.DMA,
        ],
    )
    def emb_kernel(table_hbm, x_hbm, pe_hbm, out_hbm,
                   idx_v, rows_v, pe_v, g0, g1, o0, o1, psem):
        gsem = (g0, g1)
        osem = (o0, o1)
        wid = lax.axis_index("s") * NC + lax.axis_index("c")
        s0 = wid * spw

        pdma = pltpu.async_copy(pe_hbm.at[pl.ds(s0, spw)], pe_v, psem)
        for b in range(batch):
            pltpu.sync_copy(x_hbm.at[b, pl.ds(s0, spw)], idx_v.at[b])

        chunks = [(b, h) for b in range(batch) for h in range(hpw)]

        def gather(c, buf):
            b, h = chunks[c]
            return pltpu.async_copy(
                table_hbm.at[idx_v.at[b, pl.ds(h * R, R)]], rows_v.at[buf],
                gsem[buf])

        gd = [None] * nch
        od = [None] * nch
        gd[0] = gather(0, 0)
        pdma.wait()
        for c in range(nch):
            b, h = chunks[c]
            bb = c & 1
            nb = bb ^ 1
            if c + 1 < nch:
                if c >= 1:
                    od[c - 1].wait()
                gd[c + 1] = gather(c + 1, nb)
            gd[c].wait()

            def comp(i, carry, _bb=bb, _h=h):
                for j in range(d // LANES):
                    sl = pl.ds(j * LANES, LANES)
                    rows_v[_bb, i, sl] = (
                        rows_v[_bb, i, sl] * SCALE + pe_v[_h * R + i, sl])
                return carry

            lax.fori_loop(0, R, comp, 0)
            od[c] = pltpu.async_copy(
                rows_v.at[bb], out_hbm.at[b, pl.ds(s0 + h * R, R)], osem[bb])
        if nch >= 2:
            od[nch - 2].wait()
        od[nch - 1].wait()

    return emb_kernel


def kernel(x, table):
    batch, seq_len = x.shape
    d = table.shape[1]
    pe = jnp.asarray(_make_pe_np(MAX_SEQ, d)[:seq_len])
    return _build(batch, seq_len, d)(table, x.astype(jnp.int32), pe)
